# baseline (speedup 1.0000x reference)
import functools

import jax
import jax.numpy as jnp
from jax import lax
from jax.experimental import pallas as pl
from jax.experimental.pallas import tpu as pltpu
from jax.experimental.pallas import tpu_sc as plsc

EMBED_DIM = 64
NUM_CORES = 2
NUM_SUBCORES = 16
NUM_WORKERS = NUM_CORES * NUM_SUBCORES
CHUNK = 160
NBUF = 2


@functools.cache
def _make_gather(num_rows: int):
    assert num_rows % (NUM_WORKERS * CHUNK) == 0
    rows_per_worker = num_rows // NUM_WORKERS
    n_chunks = rows_per_worker // CHUNK
    assert n_chunks % NBUF == 0
    mesh = plsc.VectorSubcoreMesh(core_axis_name="c", subcore_axis_name="s")

    @functools.partial(
        pl.kernel,
        mesh=mesh,
        compiler_params=pltpu.CompilerParams(needs_layout_passes=False),
        out_type=jax.ShapeDtypeStruct((num_rows, EMBED_DIM), jnp.float32),
        scratch_types=[
            pltpu.VMEM((rows_per_worker,), jnp.int32),
            pltpu.VMEM((256,), jnp.int32),
            pltpu.VMEM((256,), jnp.int32),
            pltpu.VMEM((NBUF, CHUNK, 2 * EMBED_DIM), jnp.float32),
            pltpu.VMEM((NBUF, CHUNK, EMBED_DIM), jnp.float32),
            pltpu.SemaphoreType.DMA((NBUF,)),
            pltpu.SemaphoreType.DMA((NBUF,)),
        ],
    )
    def gather_kernel(idx_hbm, table2_hbm, out_hbm, idx_v, pidx_a, pidx_b,
                      pairs_v, rows_v, gsem, osem):
        pidx_refs = (pidx_a, pidx_b)
        wid = lax.axis_index("s") * NUM_CORES + lax.axis_index("c")
        base = wid * rows_per_worker
        pltpu.sync_copy(idx_hbm.at[pl.ds(base, rows_per_worker)], idx_v)

        iota = lax.iota(jnp.int32, 16)

        def pidx_fill(i, slot):
            for g in range(CHUNK // 16):
                v = idx_v[pl.ds(i * CHUNK + g * 16, 16)]
                pidx_refs[slot][pl.ds(g * 16, 16)] = (
                    lax.shift_right_logical(v, 1))

        def gather_start(i, slot):
            pltpu.async_copy(
                table2_hbm.at[pidx_refs[slot].at[pl.ds(0, CHUNK)]],
                pairs_v.at[slot],
                gsem.at[slot],
            )

        def gather_wait(slot):
            pltpu.make_async_copy(
                table2_hbm.at[pidx_refs[slot].at[pl.ds(0, CHUNK)]],
                pairs_v.at[slot],
                gsem.at[slot],
            ).wait()

        def out_start(i, slot):
            pltpu.async_copy(
                rows_v.at[slot],
                out_hbm.at[pl.ds(base + i * CHUNK, CHUNK)],
                osem.at[slot],
            )

        def out_wait(slot):
            pltpu.make_async_copy(
                rows_v.at[slot],
                out_hbm.at[pl.ds(base, CHUNK)],
                osem.at[slot],
            ).wait()

        def compact(i, slot):
            src = pairs_v.at[slot]
            dst = rows_v.at[slot]
            for g in range(CHUNK // 16):
                idx16 = idx_v[pl.ds(i * CHUNK + g * 16, 16)]
                h64 = (idx16 & 1) * EMBED_DIM
                rowv = iota + (g * 16)
                for k in range(EMBED_DIM):
                    x = plsc.load_gather(src, [rowv, h64 + k])
                    plsc.store_scatter(
                        dst, [rowv, jnp.full((16,), k, jnp.int32)], x)

        pidx_fill(0, 0)
        gather_start(0, 0)

        def step(i2, carry):
            for u in range(NBUF):
                i = i2 * NBUF + u
                slot = u
                nxt = (u + 1) % NBUF

                @pl.when(i + 1 < n_chunks)
                def _():
                    pidx_fill(i + 1, nxt)
                    gather_start(i + 1, nxt)

                gather_wait(slot)

                @pl.when(i >= NBUF)
                def _():
                    out_wait(slot)

                compact(i, slot)
                out_start(i, slot)
            return carry

        lax.fori_loop(0, n_chunks // NBUF, step, 0)
        for s in range(NBUF):
            out_wait(s)

    return gather_kernel


def kernel(input_x, table):
    batch, hist = input_x.shape
    vocab = table.shape[0]
    idx = input_x.reshape(-1).astype(jnp.int32)
    table2 = table.reshape(vocab // 2, 2 * EMBED_DIM)
    y = _make_gather(idx.shape[0])(idx, table2)
    y = y.reshape(batch, hist, EMBED_DIM)
    return (y, y)


# speedup vs baseline: 3.0912x; 3.0912x over previous
import functools

import jax
import jax.numpy as jnp
from jax import lax
from jax.experimental import pallas as pl
from jax.experimental.pallas import tpu as pltpu
from jax.experimental.pallas import tpu_sc as plsc

EMBED_DIM = 64
NUM_CORES = 2
NUM_SUBCORES = 16
NUM_WORKERS = NUM_CORES * NUM_SUBCORES
CHUNK = 160
NBUF = 2

VOCAB = 100000
VHALF = VOCAB // 2
VH2 = 51200
PAIRS_PER_W = VH2 // NUM_WORKERS
BCH = 80
N_BCH = PAIRS_PER_W // BCH


@functools.cache
def _make_build():
    mesh = plsc.VectorSubcoreMesh(core_axis_name="c", subcore_axis_name="s")

    @functools.partial(
        pl.kernel,
        mesh=mesh,
        out_type=jax.ShapeDtypeStruct((2 * VH2, 2 * EMBED_DIM), jnp.float32),
        scratch_types=[
            pltpu.VMEM((2 * BCH, EMBED_DIM), jnp.float32),
            pltpu.VMEM((8, EMBED_DIM), jnp.float32),
            pltpu.VMEM((BCH, 2 * EMBED_DIM), jnp.float32),
            pltpu.VMEM((BCH, 2 * EMBED_DIM), jnp.float32),
        ],
    )
    def build_kernel(table_hbm, tables_hbm, slab_v, tiny_v, outa_v, outb_v):
        wid = lax.axis_index("s") * NUM_CORES + lax.axis_index("c")
        pbase = wid * PAIRS_PER_W

        def step(c, carry):
            p0 = pbase + c * BCH

            @pl.when(p0 < VHALF)
            def _():
                r0 = 2 * p0
                pltpu.sync_copy(table_hbm.at[pl.ds(r0, 2 * BCH)], slab_v)

                @pl.when(r0 + 2 * BCH < VOCAB)
                def _():
                    pltpu.sync_copy(
                        table_hbm.at[pl.ds(r0 + 2 * BCH, 8)], tiny_v)

                for j in range(BCH):
                    for cc in range(EMBED_DIM // 16):
                        s = pl.ds(cc * 16, 16)
                        s2 = pl.ds(EMBED_DIM + cc * 16, 16)
                        va = slab_v[2 * j, s]
                        vb = slab_v[2 * j + 1, s]
                        vc = slab_v[2 * j + 2, s] if j + 1 < BCH else (
                            tiny_v[0, s])
                        outa_v[j, s] = va
                        outa_v[j, s2] = vb
                        outb_v[j, s] = vb
                        outb_v[j, s2] = vc

                pltpu.sync_copy(outa_v, tables_hbm.at[pl.ds(p0, BCH)])
                pltpu.sync_copy(outb_v, tables_hbm.at[pl.ds(VH2 + p0, BCH)])

            return carry

        lax.fori_loop(0, N_BCH, step, 0)

    return build_kernel


@functools.cache
def _make_gather(num_rows: int):
    assert num_rows % (NUM_WORKERS * CHUNK) == 0
    rows_per_worker = num_rows // NUM_WORKERS
    n_chunks = rows_per_worker // CHUNK
    assert n_chunks % NBUF == 0
    mesh = plsc.VectorSubcoreMesh(core_axis_name="c", subcore_axis_name="s")

    @functools.partial(
        pl.kernel,
        mesh=mesh,
        out_type=jax.ShapeDtypeStruct((num_rows, EMBED_DIM), jnp.float32),
        scratch_types=[
            pltpu.VMEM((rows_per_worker,), jnp.int32),
            pltpu.VMEM((NBUF, CHUNK, 2 * EMBED_DIM), jnp.float32),
            pltpu.VMEM((NBUF, CHUNK, EMBED_DIM), jnp.float32),
            pltpu.SemaphoreType.DMA((NBUF,)),
            pltpu.SemaphoreType.DMA((NBUF,)),
        ],
    )
    def gather_kernel(idx_hbm, tables_hbm, out_hbm, idx_v, pairs_v, rows_v,
                      gsem, osem):
        wid = lax.axis_index("s") * NUM_CORES + lax.axis_index("c")
        base = wid * rows_per_worker
        pltpu.sync_copy(idx_hbm.at[pl.ds(base, rows_per_worker)], idx_v)

        def gather_start(i, slot):
            pltpu.async_copy(
                tables_hbm.at[idx_v.at[pl.ds(i * CHUNK, CHUNK)]],
                pairs_v.at[slot],
                gsem.at[slot],
            )

        def gather_wait(slot):
            pltpu.make_async_copy(
                tables_hbm.at[idx_v.at[pl.ds(0, CHUNK)]],
                pairs_v.at[slot],
                gsem.at[slot],
            ).wait()

        def out_start(i, slot):
            pltpu.async_copy(
                rows_v.at[slot],
                out_hbm.at[pl.ds(base + i * CHUNK, CHUNK)],
                osem.at[slot],
            )

        def out_wait(slot):
            pltpu.make_async_copy(
                rows_v.at[slot],
                out_hbm.at[pl.ds(base, CHUNK)],
                osem.at[slot],
            ).wait()

        def compact(slot):
            for r in range(CHUNK):
                for c in range(EMBED_DIM // 16):
                    rows_v[slot, r, pl.ds(c * 16, 16)] = (
                        pairs_v[slot, r, pl.ds(c * 16, 16)])

        gather_start(0, 0)

        def step(i2, carry):
            for u in range(NBUF):
                i = i2 * NBUF + u
                slot = u
                nxt = (u + 1) % NBUF

                @pl.when(i + 1 < n_chunks)
                def _():
                    gather_start(i + 1, nxt)

                gather_wait(slot)

                @pl.when(i >= NBUF)
                def _():
                    out_wait(slot)

                compact(slot)
                out_start(i, slot)
            return carry

        lax.fori_loop(0, n_chunks // NBUF, step, 0)
        for s in range(NBUF):
            out_wait(s)

    return gather_kernel


def kernel(input_x, table):
    batch, hist = input_x.shape
    idx = input_x.reshape(-1).astype(jnp.int32)
    slots = (idx >> 1) + (idx & 1) * VH2
    tables = _make_build()(table)
    y = _make_gather(idx.shape[0])(slots, tables)
    y = y.reshape(batch, hist, EMBED_DIM)
    return (y, y)
